# Initial kernel scaffold; baseline (speedup 1.0000x reference)
import jax
import jax.numpy as jnp
from jax.experimental import pallas as pl


def kernel(xyz):
    raise NotImplementedError("write your pallas kernel here")


# speedup vs baseline: 11.4080x; 11.4080x over previous
import functools

import jax
import jax.numpy as jnp
from jax import lax
from jax.experimental import pallas as pl
from jax.experimental.pallas import tpu as pltpu
from jax.experimental.pallas import tpu_sc as plsc

NUM_GROUP = 512
GROUP_SIZE = 32

_NC = 2
_NS = 16
_NW = _NC * _NS
_L = 16


def _fps_body(xt_ref, cxo_ref, cyo_ref, czo_ref, md_ref):
    B = xt_ref.shape[1]
    N = xt_ref.shape[2]
    G = cxo_ref.shape[0]
    x = xt_ref[0]
    y = xt_ref[1]
    z = xt_ref[2]
    md_ref[...] = jnp.full((B, N), 1e10, jnp.float32)
    iota = lax.broadcasted_iota(jnp.int32, (B, N), 1)
    zero = jnp.zeros((B, N), jnp.float32)

    def step(i, far):
        eq = iota == far
        cx = jnp.sum(jnp.where(eq, x, zero), axis=1, keepdims=True)
        cy = jnp.sum(jnp.where(eq, y, zero), axis=1, keepdims=True)
        cz = jnp.sum(jnp.where(eq, z, zero), axis=1, keepdims=True)
        cxo_ref[pl.ds(i, 1), :] = jnp.reshape(cx, (1, B))
        cyo_ref[pl.ds(i, 1), :] = jnp.reshape(cy, (1, B))
        czo_ref[pl.ds(i, 1), :] = jnp.reshape(cz, (1, B))
        dx = x - cx
        dy = y - cy
        dz = z - cz
        d = (dx * dx + dy * dy) + dz * dz
        md = jnp.minimum(md_ref[...], d)
        md_ref[...] = md
        m = jnp.max(md, axis=1, keepdims=True)
        far_new = jnp.min(jnp.where(md == m, iota, N), axis=1, keepdims=True)
        return far_new

    lax.fori_loop(0, G, step, jnp.zeros((B, 1), jnp.int32))


def _fps(xt, B, N, G):
    out = pl.pallas_call(
        _fps_body,
        out_shape=[
            jax.ShapeDtypeStruct((G, B), jnp.float32),
            jax.ShapeDtypeStruct((G, B), jnp.float32),
            jax.ShapeDtypeStruct((G, B), jnp.float32),
        ],
        scratch_shapes=[pltpu.VMEM((B, N), jnp.float32)],
    )(xt)
    return out


def _knn_body(xt_ref, cx_ref, cy_ref, cz_ref, out_ref, d_ref):
    N = xt_ref.shape[2]
    G = cx_ref.shape[2]
    K = out_ref.shape[1]
    x = xt_ref[0, 0:1, :]
    y = xt_ref[0, 1:2, :]
    z = xt_ref[0, 2:3, :]
    cx = jnp.reshape(cx_ref[0, 0, :], (G, 1))
    cy = jnp.reshape(cy_ref[0, 0, :], (G, 1))
    cz = jnp.reshape(cz_ref[0, 0, :], (G, 1))
    bf = jnp.bfloat16
    f32 = jnp.float32
    bx = x.astype(bf).astype(f32)
    by = y.astype(bf).astype(f32)
    bz = z.astype(bf).astype(f32)
    bcx = cx.astype(bf).astype(f32)
    bcy = cy.astype(bf).astype(f32)
    bcz = cz.astype(bf).astype(f32)
    sq_x = (x * x + y * y) + z * z
    sq_c = (cx * cx + cy * cy) + cz * cz
    inner = (bcx * bx + bcy * by) + bcz * bz
    d_ref[...] = (sq_c - 2.0 * inner) + sq_x
    iota = lax.broadcasted_iota(jnp.int32, (G, N), 1)

    def sel(k, carry):
        d = d_ref[...]
        m = jnp.min(d, axis=1, keepdims=True)
        j = jnp.min(jnp.where(d == m, iota, N), axis=1, keepdims=True)
        out_ref[0, pl.ds(k, 1), :] = jnp.reshape(j, (1, G))
        d_ref[...] = jnp.where(iota == j, jnp.inf, d)
        return carry

    lax.fori_loop(0, K, sel, 0)


def _knn(xtb, cxt, cyt, czt, B, N, G, K):
    return pl.pallas_call(
        _knn_body,
        grid=(B,),
        in_specs=[
            pl.BlockSpec((1, 3, N), lambda b: (b, 0, 0)),
            pl.BlockSpec((1, 1, G), lambda b: (b, 0, 0)),
            pl.BlockSpec((1, 1, G), lambda b: (b, 0, 0)),
            pl.BlockSpec((1, 1, G), lambda b: (b, 0, 0)),
        ],
        out_specs=pl.BlockSpec((1, K, G), lambda b: (b, 0, 0)),
        out_shape=jax.ShapeDtypeStruct((B, K, G), jnp.int32),
        scratch_shapes=[pltpu.VMEM((G, N), jnp.float32)],
    )(xtb, cxt, cyt, czt)


def _make_gather(B, N, G, K):
    P = B * G * K
    M = P // _NW
    R = M // 128
    PR = P // 128
    mesh = plsc.VectorSubcoreMesh(core_axis_name="c", subcore_axis_name="s")

    @functools.partial(
        pl.kernel,
        mesh=mesh,
        out_type=jax.ShapeDtypeStruct((3, PR, 128), jnp.float32),
        scratch_types=[
            pltpu.VMEM((R, 128), jnp.int32),
            pltpu.VMEM((R, 128), jnp.int32),
            pltpu.VMEM((R, 128), jnp.float32),
            pltpu.VMEM((R, 128), jnp.float32),
            pltpu.SemaphoreType.DMA,
        ],
    )
    def gather_k(tbl_hbm, idx_hbm, cen_hbm, out_hbm, idx_v, eidx_v, gat_v,
                 cen_v, sem):
        cid = lax.axis_index("c")
        sid = lax.axis_index("s")
        wid = sid * _NC + cid
        row0 = wid * R
        b = (wid * M) // (G * K)
        base3 = 3 * b * N
        pltpu.sync_copy(idx_hbm.at[pl.ds(row0, R)], idx_v)

        for c in range(3):
            def mk(t, carry, _c=c):
                r = t // 8
                l = (t % 8) * _L
                v = idx_v[r, pl.ds(l, _L)]
                eidx_v[r, pl.ds(l, _L)] = v * 3 + (base3 + _c)
                return carry

            lax.fori_loop(0, R * 8, mk, 0)

            def fire(j, carry):
                pltpu.make_async_copy(
                    tbl_hbm.at[eidx_v.at[j]], gat_v.at[j], sem).start()
                return carry

            lax.fori_loop(0, R, fire, 0)

            def drain(j, carry):
                pltpu.make_async_copy(
                    tbl_hbm.at[eidx_v.at[j]], gat_v.at[j], sem).wait()
                return carry

            lax.fori_loop(0, R, drain, 0)

            pltpu.sync_copy(cen_hbm.at[c, pl.ds(row0, R)], cen_v)

            def sub(t, carry):
                r = t // 8
                l = (t % 8) * _L
                gat_v[r, pl.ds(l, _L)] = (
                    gat_v[r, pl.ds(l, _L)] - cen_v[r, pl.ds(l, _L)])
                return carry

            lax.fori_loop(0, R * 8, sub, 0)

            pltpu.sync_copy(gat_v, out_hbm.at[c, pl.ds(row0, R)])

    return gather_k


def kernel(xyz):
    B, N, _ = xyz.shape
    G, K = NUM_GROUP, GROUP_SIZE
    P = B * G * K
    xt = jnp.transpose(xyz, (2, 0, 1))
    cxo, cyo, czo = _fps(xt, B, N, G)
    cxt = jnp.transpose(cxo).reshape(B, 1, G)
    cyt = jnp.transpose(cyo).reshape(B, 1, G)
    czt = jnp.transpose(czo).reshape(B, 1, G)
    xtb = jnp.transpose(xyz, (0, 2, 1))
    outk = _knn(xtb, cxt, cyt, czt, B, N, G, K)
    idx = jnp.transpose(outk, (0, 2, 1))
    center = jnp.stack(
        [jnp.transpose(cxo), jnp.transpose(cyo), jnp.transpose(czo)], axis=-1)

    tbl = xyz.reshape(B * N * 3)
    idx2d = idx.reshape(P // 128, 128)
    cen_rep = jnp.broadcast_to(center[:, :, None, :], (B, G, K, 3))
    cenp = jnp.transpose(cen_rep, (3, 0, 1, 2)).reshape(3, P // 128, 128)
    outp = _make_gather(B, N, G, K)(tbl, idx2d, cenp)
    neigh = jnp.transpose(outp.reshape(3, B, G, K), (1, 2, 3, 0))
    return (neigh, center)
